# baseline (speedup 1.0000x reference)
import functools

import jax
import jax.numpy as jnp
from jax import lax
from jax.experimental import pallas as pl
from jax.experimental.pallas import tpu as pltpu
from jax.experimental.pallas import tpu_sc as plsc

N = 10000
E = 320000
D = 128
H = 128
C = 16

NC = 2
NS = 16
NW = NC * NS
EPT = E // NW
K = 16
GK = 5
NG = EPT // (K * GK)
NPAD = 10240
RPT = NPAD // NS
CPT = NPAD // NS
BPW = NPAD // NW

_SQRT_HALF = 0.7071067811865476


def _gelu(x):
    return x * 0.5 * (1.0 + lax.erf(x * _SQRT_HALF))


BM = 1000


def _dot(a, b):
    return jnp.dot(a, b, preferred_element_type=jnp.float32,
                   precision=lax.Precision.HIGHEST)


def _tc_pre_body(nf, s1, t1, w1, b1, s2, t2, w2, b2, x_out, y_out):
    x = _gelu(_dot(nf[...] * s1[...] + t1[...], w1[...]) + b1[...])
    x_out[...] = x
    y_out[...] = _gelu(_dot(x * s2[...] + t2[...], w2[...]) + b2[...])


def _tc_upd_body(x, ps, pc, sa, ta, sb, tb, wa, wb, bu, s2, t2, w2, b2,
                 x1_out, y2_out):
    sums = ps[0] + ps[1]
    cnt = jnp.maximum(pc[0] + pc[1], 1.0)
    agg = sums / cnt
    h = (_dot(x[...] * sa[...] + ta[...], wa[...])
         + _dot(agg * sb[...] + tb[...], wb[...]) + bu[...])
    x1 = _gelu(h)
    x1_out[...] = x1
    y2_out[...] = _gelu(_dot(x1 * s2[...] + t2[...], w2[...]) + b2[...])


def _tc_fin_body(x, ps, pc, sa, ta, sb, tb, wa, wb, bu, sp, tp, wp, bp,
                 ow, ob, z_out):
    sums = ps[0] + ps[1]
    cnt = jnp.maximum(pc[0] + pc[1], 1.0)
    agg = sums / cnt
    h = (_dot(x[...] * sa[...] + ta[...], wa[...])
         + _dot(agg * sb[...] + tb[...], wb[...]) + bu[...])
    x2 = _gelu(h)
    xp = _gelu(_dot(x2 * sp[...] + tp[...], wp[...]) + bp[...])
    z_out[...] = _dot(xp, ow[...]) + ob[...]


def _vec_spec(n):
    return pl.BlockSpec((n,), lambda i: (0,))


def _mat_spec(r, c):
    return pl.BlockSpec((r, c), lambda i: (0, 0))


def _row_spec(c):
    return pl.BlockSpec((BM, c), lambda i: (i, 0))


def _ps_spec():
    return pl.BlockSpec((2, BM, D), lambda i: (0, i, 0))


def _pc_spec():
    return pl.BlockSpec((2, BM, 1), lambda i: (0, i, 0))


_f32 = jnp.float32

_tc_pre = pl.pallas_call(
    _tc_pre_body,
    grid=(N // BM,),
    in_specs=[_row_spec(D),
              _vec_spec(D), _vec_spec(D), _mat_spec(D, H), _vec_spec(H),
              _vec_spec(H), _vec_spec(H), _mat_spec(H, H), _vec_spec(H)],
    out_specs=[_row_spec(H), _row_spec(H)],
    out_shape=[jax.ShapeDtypeStruct((N, H), _f32),
               jax.ShapeDtypeStruct((N, H), _f32)],
)

_tc_upd = pl.pallas_call(
    _tc_upd_body,
    grid=(N // BM,),
    in_specs=[_row_spec(H), _ps_spec(), _pc_spec(),
              _vec_spec(H), _vec_spec(H), _vec_spec(H), _vec_spec(H),
              _mat_spec(H, H), _mat_spec(H, H), _vec_spec(H),
              _vec_spec(H), _vec_spec(H), _mat_spec(H, H), _vec_spec(H)],
    out_specs=[_row_spec(H), _row_spec(H)],
    out_shape=[jax.ShapeDtypeStruct((N, H), _f32),
               jax.ShapeDtypeStruct((N, H), _f32)],
)

_tc_fin = pl.pallas_call(
    _tc_fin_body,
    grid=(N // BM,),
    in_specs=[_row_spec(H), _ps_spec(), _pc_spec(),
              _vec_spec(H), _vec_spec(H), _vec_spec(H), _vec_spec(H),
              _mat_spec(H, H), _mat_spec(H, H), _vec_spec(H),
              _vec_spec(H), _vec_spec(H), _mat_spec(H, H), _vec_spec(H),
              _mat_spec(H, H), _vec_spec(H)],
    out_specs=[_row_spec(H)],
    out_shape=[jax.ShapeDtypeStruct((N, H), _f32)],
)


_MESH = plsc.VectorSubcoreMesh(core_axis_name="c", subcore_axis_name="s")

EP = 327680
EPT2 = EP // NW
KG = 64
NCH = EPT2 // KG
NPAIR = NCH // 2
SUB = KG // 16
DRT = EPT2 // 16
KF = 80


def _make_agg(with_cnt):
    out_type = [jax.ShapeDtypeStruct((NC, NS, RPT, D), _f32)]
    scratch = [
        pltpu.VMEM_SHARED((NPAD, D), _f32),
        pltpu.VMEM((EPT2,), jnp.int32),
        pltpu.VMEM((8, 16), jnp.int32),
        pltpu.VMEM((8, 16), jnp.int32),
        pltpu.VMEM((8, 16), jnp.int32),
        pltpu.VMEM((8, 16), jnp.int32),
        pltpu.VMEM((KG, D), _f32),
        pltpu.VMEM((KG, D), _f32),
        pltpu.VMEM((KG, D), _f32),
        pltpu.VMEM((KG, D), _f32),
        pltpu.SemaphoreType.DMA,
        pltpu.SemaphoreType.DMA,
        pltpu.SemaphoreType.DMA,
        pltpu.SemaphoreType.DMA,
        pltpu.SemaphoreType.DMA,
    ]
    if with_cnt:
        out_type.append(jax.ShapeDtypeStruct((NC, NS, CPT), _f32))
        scratch.append(pltpu.VMEM_SHARED((NPAD,), _f32))
        scratch.append(pltpu.VMEM((16,), _f32))

    def body(*refs):
        if with_cnt:
            (y_hbm, dst_hbm, src_hbm, zrows_hbm, zcnt_hbm,
             psum_out, pcnt_out, acc, src_v, d0, d1, d2, d3,
             r0_, r1_, r2_, r3_, s0, s1, s2, s3, sem_s, cacc, ones_v) = refs
        else:
            (y_hbm, dst_hbm, src_hbm, zrows_hbm,
             psum_out, acc, src_v, d0, d1, d2, d3,
             r0_, r1_, r2_, r3_, s0, s1, s2, s3, sem_s) = refs
        cid = lax.axis_index("c")
        sid = lax.axis_index("s")
        wid = cid * NS + sid

        r0 = sid * RPT
        pltpu.sync_copy(zrows_hbm.at[pl.ds(r0, RPT)], acc.at[pl.ds(r0, RPT)])
        if with_cnt:
            c0 = sid * CPT
            pltpu.sync_copy(zcnt_hbm.at[pl.ds(c0, CPT)], cacc.at[pl.ds(c0, CPT)])
            ones_v[...] = jnp.ones((16,), _f32)
        pltpu.sync_copy(src_hbm.at[pl.ds(wid * EPT2, EPT2)], src_v)
        plsc.subcore_barrier()

        def fire(c, dbuf, buf, sem):
            pltpu.async_copy(dst_hbm.at[wid * NCH + c], dbuf, sem)
            pltpu.async_copy(y_hbm.at[src_v.at[pl.ds(c * KG, KG)]], buf, sem)

        def drain(dbuf, buf, sem):
            pltpu.make_async_copy(dst_hbm.at[0], dbuf, sem).wait()
            pltpu.make_async_copy(
                y_hbm.at[src_v.at[pl.ds(0, KG)]], buf, sem).wait()

        def scatter(dbuf, buf):
            descs = []
            for s in range(SUB):
                descs.append(pltpu.async_copy(
                    buf.at[pl.ds(s * 16, 16)], acc.at[dbuf.at[s]], sem_s,
                    add=True))
                if with_cnt:
                    descs.append(pltpu.async_copy(
                        ones_v, cacc.at[dbuf.at[s]], sem_s, add=True))
            for d in descs:
                d.wait()

        bufs = [(d0, r0_, s0), (d1, r1_, s1), (d2, r2_, s2), (d3, r3_, s3)]
        for b in range(4):
            fire(b, *bufs[b])

        def quad(q, carry):
            c0 = 4 * q
            for b in range(4):
                dbuf, buf, sem = bufs[b]
                drain(dbuf, buf, sem)
                scatter(dbuf, buf)

                @pl.when(q < NCH // 4 - 1)
                def _(dbuf=dbuf, buf=buf, sem=sem, c=c0 + b):
                    fire(c + 4, dbuf, buf, sem)

            return carry

        lax.fori_loop(0, NCH // 4, quad, 0)
        plsc.subcore_barrier()

        pltpu.sync_copy(acc.at[pl.ds(r0, RPT)], psum_out.at[cid, sid])
        if with_cnt:
            pltpu.sync_copy(cacc.at[pl.ds(c0, CPT)], pcnt_out.at[cid, sid])

    return pl.kernel(body, out_type=out_type, mesh=_MESH,
                     scratch_types=scratch)


_agg_cnt = _make_agg(True)
_agg = _make_agg(False)


@functools.partial(
    pl.kernel,
    out_type=jax.ShapeDtypeStruct((NPAD, H), _f32),
    mesh=_MESH,
    scratch_types=[
        pltpu.VMEM((BPW,), jnp.int32),
        pltpu.VMEM((BPW // KF, KF, H), _f32),
        pltpu.SemaphoreType.DMA,
    ],
)
def _final_gather(z_hbm, idx_hbm, out_hbm, idx_v, rows_v, sem):
    wid = lax.axis_index("c") * NS + lax.axis_index("s")
    base = wid * BPW
    pltpu.sync_copy(idx_hbm.at[pl.ds(base, BPW)], idx_v)
    descs = []
    for j in range(BPW // KF):
        descs.append(
            pltpu.async_copy(z_hbm.at[idx_v.at[pl.ds(j * KF, KF)]],
                             rows_v.at[j], sem))
    for j in range(BPW // KF):
        descs[j].wait()
        pltpu.sync_copy(rows_v.at[j], out_hbm.at[pl.ds(base + j * KF, KF)])


def _fold_bn(g, b, m, v):
    s = g / jnp.sqrt(v + 1e-5)
    return s, b - m * s


def kernel(input_node_indices, node_features, edge_index,
           pre_g, pre_b, pre_m, pre_v, pre_w, pre_bias,
           c1p_g, c1p_b, c1p_m, c1p_v, c1p_w, c1p_bias,
           c1u_g, c1u_b, c1u_m, c1u_v, c1u_w, c1u_bias,
           c2p_g, c2p_b, c2p_m, c2p_v, c2p_w, c2p_bias,
           c2u_g, c2u_b, c2u_m, c2u_v, c2u_w, c2u_bias,
           post_g, post_b, post_m, post_v, post_w, post_bias,
           out_w, out_b):
    dst = edge_index[0]
    src = edge_index[1]
    dstp = jnp.concatenate([dst, jnp.full((EP - E,), N + 200, jnp.int32)])
    dst2 = jnp.pad(dstp.reshape(EP // KG, SUB, 16), ((0, 0), (0, 8 - SUB), (0, 0)),
                   constant_values=N + 200)
    srcp = jnp.concatenate([src, jnp.zeros((EP - E,), jnp.int32)])

    s_pre, t_pre = _fold_bn(pre_g, pre_b, pre_m, pre_v)
    s_c1p, t_c1p = _fold_bn(c1p_g, c1p_b, c1p_m, c1p_v)
    s_c1u, t_c1u = _fold_bn(c1u_g, c1u_b, c1u_m, c1u_v)
    s_c2p, t_c2p = _fold_bn(c2p_g, c2p_b, c2p_m, c2p_v)
    s_c2u, t_c2u = _fold_bn(c2u_g, c2u_b, c2u_m, c2u_v)
    s_post, t_post = _fold_bn(post_g, post_b, post_m, post_v)

    zrows = jnp.zeros((NPAD, D), _f32)
    zcnt = jnp.zeros((NPAD,), _f32)

    x, y1 = _tc_pre(node_features, s_pre, t_pre, pre_w, pre_bias,
                    s_c1p, t_c1p, c1p_w, c1p_bias)

    ps1, pc = _agg_cnt(y1, dst2, srcp, zrows, zcnt)
    ps1 = ps1.reshape(2, NPAD, D)[:, :N]
    pcr = pc.reshape(2, NPAD)[:, :N, None]

    x1, y2 = _tc_upd(x, ps1, pcr,
                     s_c1u[:H], t_c1u[:H], s_c1u[H:], t_c1u[H:],
                     c1u_w[:H], c1u_w[H:], c1u_bias,
                     s_c2p, t_c2p, c2p_w, c2p_bias)

    (ps2,) = _agg(y2, dst2, srcp, zrows)
    ps2 = ps2.reshape(2, NPAD, D)[:, :N]

    ow_pad = jnp.zeros((H, H), _f32).at[:, :C].set(out_w)
    ob_pad = jnp.zeros((H,), _f32).at[:C].set(out_b)
    (z,) = _tc_fin(x1, ps2, pcr,
                   s_c2u[:H], t_c2u[:H], s_c2u[H:], t_c2u[H:],
                   c2u_w[:H], c2u_w[H:], c2u_bias,
                   s_post, t_post, post_w, post_bias,
                   ow_pad, ob_pad)

    idx = jnp.concatenate(
        [input_node_indices.astype(jnp.int32),
         jnp.zeros((NPAD - N,), jnp.int32)])
    return _final_gather(z, idx)[:N, :C]


# speedup vs baseline: 1.0155x; 1.0155x over previous
import functools

import jax
import jax.numpy as jnp
from jax import lax
from jax.experimental import pallas as pl
from jax.experimental.pallas import tpu as pltpu
from jax.experimental.pallas import tpu_sc as plsc

N = 10000
E = 320000
D = 128
H = 128
C = 16

NC = 2
NS = 16
NW = NC * NS
EPT = E // NW
K = 16
GK = 5
NG = EPT // (K * GK)
NPAD = 10240
RPT = NPAD // NS
CPT = NPAD // NS
BPW = NPAD // NW

_SQRT_HALF = 0.7071067811865476


def _gelu(x):
    return x * 0.5 * (1.0 + lax.erf(x * _SQRT_HALF))


BM = 1024


def _dot(a, b):
    return jnp.dot(a, b, preferred_element_type=jnp.float32,
                   precision=lax.Precision.HIGHEST)


def _tc_pre_body(nf, s1, t1, w1, b1, s2, t2, w2, b2, x_out, y_out):
    x = _gelu(_dot(nf[...] * s1[...] + t1[...], w1[...]) + b1[...])
    x_out[...] = x
    y = _gelu(_dot(x * s2[...] + t2[...], w2[...]) + b2[...])
    y_out[...] = y.T.reshape(32, 4, y.shape[0])


def _tc_upd_body(x, ps, pc, sa, ta, sb, tb, wa, wb, bu, s2, t2, w2, b2,
                 x1_out, y2_out):
    sums = ps[...].T
    cnt = jnp.maximum(pc[...], 1.0)
    agg = sums / cnt
    h = (_dot(x[...] * sa[...] + ta[...], wa[...])
         + _dot(agg * sb[...] + tb[...], wb[...]) + bu[...])
    x1 = _gelu(h)
    x1_out[...] = x1
    y2 = _gelu(_dot(x1 * s2[...] + t2[...], w2[...]) + b2[...])
    y2_out[...] = y2.T.reshape(32, 4, y2.shape[0])


def _tc_fin_body(x, ps, pc, sa, ta, sb, tb, wa, wb, bu, sp, tp, wp, bp,
                 ow, ob, z_out):
    sums = ps[...].T
    cnt = jnp.maximum(pc[...], 1.0)
    agg = sums / cnt
    h = (_dot(x[...] * sa[...] + ta[...], wa[...])
         + _dot(agg * sb[...] + tb[...], wb[...]) + bu[...])
    x2 = _gelu(h)
    xp = _gelu(_dot(x2 * sp[...] + tp[...], wp[...]) + bp[...])
    z_out[...] = _dot(xp, ow[...]) + ob[...]


def _vec_spec(n):
    return pl.BlockSpec((n,), lambda i: (0,))


def _mat_spec(r, c):
    return pl.BlockSpec((r, c), lambda i: (0, 0))


def _row_spec(c):
    return pl.BlockSpec((BM, c), lambda i: (i, 0))


def _ps_spec():
    return pl.BlockSpec((D, BM), lambda i: (0, i))


def _pc_spec():
    return pl.BlockSpec((BM, 1), lambda i: (i, 0))


def _yt_spec():
    return pl.BlockSpec((32, 4, BM), lambda i: (0, 0, i))


_f32 = jnp.float32

_tc_pre = pl.pallas_call(
    _tc_pre_body,
    grid=(NPAD // BM,),
    in_specs=[_row_spec(D),
              _vec_spec(D), _vec_spec(D), _mat_spec(D, H), _vec_spec(H),
              _vec_spec(H), _vec_spec(H), _mat_spec(H, H), _vec_spec(H)],
    out_specs=[_row_spec(H), _yt_spec()],
    out_shape=[jax.ShapeDtypeStruct((N, H), _f32),
               jax.ShapeDtypeStruct((32, 4, NPAD), _f32)],
)

_tc_upd = pl.pallas_call(
    _tc_upd_body,
    grid=(NPAD // BM,),
    in_specs=[_row_spec(H), _ps_spec(), _pc_spec(),
              _vec_spec(H), _vec_spec(H), _vec_spec(H), _vec_spec(H),
              _mat_spec(H, H), _mat_spec(H, H), _vec_spec(H),
              _vec_spec(H), _vec_spec(H), _mat_spec(H, H), _vec_spec(H)],
    out_specs=[_row_spec(H), _yt_spec()],
    out_shape=[jax.ShapeDtypeStruct((N, H), _f32),
               jax.ShapeDtypeStruct((32, 4, NPAD), _f32)],
)

_tc_fin = pl.pallas_call(
    _tc_fin_body,
    grid=(NPAD // BM,),
    in_specs=[_row_spec(H), _ps_spec(), _pc_spec(),
              _vec_spec(H), _vec_spec(H), _vec_spec(H), _vec_spec(H),
              _mat_spec(H, H), _mat_spec(H, H), _vec_spec(H),
              _vec_spec(H), _vec_spec(H), _mat_spec(H, H), _vec_spec(H),
              _mat_spec(H, H), _vec_spec(H)],
    out_specs=[_row_spec(H)],
    out_shape=[jax.ShapeDtypeStruct((N, H), _f32)],
)


_MESH = plsc.VectorSubcoreMesh(core_axis_name="c", subcore_axis_name="s")

EP = 327680
EPT2 = EP // NW
KG = 80
NCH = EPT2 // KG
NPAIR = NCH // 2
SUB = KG // 16
DRT = EPT2 // 16
KF = 80


ECH = 2048
NECH = EP // ECH
UNR = 4


def _make_agg(with_cnt):
    out_type = [jax.ShapeDtypeStruct((NW, 4, NPAD), _f32)]
    scratch = [
        pltpu.VMEM((4 * NPAD,), _f32),
        pltpu.VMEM((4 * NPAD,), _f32),
        pltpu.VMEM((ECH,), jnp.int32),
        pltpu.VMEM((ECH,), jnp.int32),
        pltpu.VMEM((ECH,), jnp.int32),
        pltpu.VMEM((ECH,), jnp.int32),
        pltpu.SemaphoreType.DMA,
        pltpu.SemaphoreType.DMA,
    ]
    if with_cnt:
        out_type.append(jax.ShapeDtypeStruct((NPAD,), _f32))
        scratch.append(pltpu.VMEM((NPAD,), _f32))

    def body(*refs):
        if with_cnt:
            (y_hbm, dst_hbm, src_hbm, zrows_hbm, zcnt_hbm,
             sum_out, cnt_out, y_loc, acc, sa_, sb_, da_, db_,
             sem_a, sem_b, cacc) = refs
        else:
            (y_hbm, dst_hbm, src_hbm, zrows_hbm,
             sum_out, y_loc, acc, sa_, sb_, da_, db_,
             sem_a, sem_b) = refs
        cid = lax.axis_index("c")
        sid = lax.axis_index("s")
        wid = cid * NS + sid

        for c in range(4):
            pltpu.sync_copy(y_hbm.at[wid, c], y_loc.at[pl.ds(c * NPAD, NPAD)])
            pltpu.sync_copy(zrows_hbm, acc.at[pl.ds(c * NPAD, NPAD)])
        if with_cnt:
            pltpu.sync_copy(zcnt_hbm, cacc)
        ones16 = jnp.ones((16,), _f32)
        coff = [jnp.full((16,), c * NPAD, jnp.int32) for c in range(4)]

        def fire(c, sbuf, dbuf, sem):
            pltpu.async_copy(src_hbm.at[pl.ds(c * ECH, ECH)], sbuf, sem)
            pltpu.async_copy(dst_hbm.at[pl.ds(c * ECH, ECH)], dbuf, sem)

        def drain(sbuf, dbuf, sem):
            pltpu.make_async_copy(src_hbm.at[pl.ds(0, ECH)], sbuf, sem).wait()
            pltpu.make_async_copy(dst_hbm.at[pl.ds(0, ECH)], dbuf, sem).wait()

        def process(sbuf, dbuf):
            def step(i, carry):
                for u in range(UNR):
                    e0 = (i * UNR + u) * 16
                    s16 = sbuf[pl.ds(e0, 16)]
                    d16 = dbuf[pl.ds(e0, 16)]
                    for c in range(4):
                        v = plsc.load_gather(y_loc, [s16 + coff[c]])
                        plsc.addupdate_scatter(acc, [d16 + coff[c]], v)
                    if with_cnt:
                        plsc.addupdate_scatter(cacc, [d16], ones16)
                return carry
            lax.fori_loop(0, ECH // (16 * UNR), step, 0)

        fire(0, sa_, da_, sem_a)

        def pair(g, carry):
            cA = 2 * g
            fire(cA + 1, sb_, db_, sem_b)
            drain(sa_, da_, sem_a)
            process(sa_, da_)

            @pl.when(g < NECH // 2 - 1)
            def _():
                fire(cA + 2, sa_, da_, sem_a)

            drain(sb_, db_, sem_b)
            process(sb_, db_)
            return carry

        lax.fori_loop(0, NECH // 2, pair, 0)

        for c in range(4):
            pltpu.sync_copy(acc.at[pl.ds(c * NPAD, NPAD)], sum_out.at[wid, c])
        if with_cnt:
            pltpu.sync_copy(cacc.at[pl.ds(wid * BPW, BPW)],
                            cnt_out.at[pl.ds(wid * BPW, BPW)])

    return pl.kernel(body, out_type=out_type, mesh=_MESH,
                     scratch_types=scratch,
                     compiler_params=pltpu.CompilerParams(
                         use_tc_tiling_on_sc=False,
                         needs_layout_passes=False))


_agg_cnt = _make_agg(True)
_agg = _make_agg(False)


@functools.partial(
    pl.kernel,
    out_type=jax.ShapeDtypeStruct((NPAD, H), _f32),
    mesh=_MESH,
    scratch_types=[
        pltpu.VMEM((BPW,), jnp.int32),
        pltpu.VMEM((BPW // KF, KF, H), _f32),
        pltpu.SemaphoreType.DMA,
    ],
)
def _final_gather(z_hbm, idx_hbm, out_hbm, idx_v, rows_v, sem):
    wid = lax.axis_index("c") * NS + lax.axis_index("s")
    base = wid * BPW
    pltpu.sync_copy(idx_hbm.at[pl.ds(base, BPW)], idx_v)
    descs = []
    for j in range(BPW // KF):
        descs.append(
            pltpu.async_copy(z_hbm.at[idx_v.at[pl.ds(j * KF, KF)]],
                             rows_v.at[j], sem))
    for j in range(BPW // KF):
        descs[j].wait()
        pltpu.sync_copy(rows_v.at[j], out_hbm.at[pl.ds(base + j * KF, KF)])


def _fold_bn(g, b, m, v):
    s = g / jnp.sqrt(v + 1e-5)
    return s, b - m * s


def kernel(input_node_indices, node_features, edge_index,
           pre_g, pre_b, pre_m, pre_v, pre_w, pre_bias,
           c1p_g, c1p_b, c1p_m, c1p_v, c1p_w, c1p_bias,
           c1u_g, c1u_b, c1u_m, c1u_v, c1u_w, c1u_bias,
           c2p_g, c2p_b, c2p_m, c2p_v, c2p_w, c2p_bias,
           c2u_g, c2u_b, c2u_m, c2u_v, c2u_w, c2u_bias,
           post_g, post_b, post_m, post_v, post_w, post_bias,
           out_w, out_b):
    dst = edge_index[0]
    src = edge_index[1]
    dstp = jnp.concatenate([dst, jnp.full((EP - E,), N + 200, jnp.int32)])
    srcp = jnp.concatenate([src, jnp.zeros((EP - E,), jnp.int32)])

    s_pre, t_pre = _fold_bn(pre_g, pre_b, pre_m, pre_v)
    s_c1p, t_c1p = _fold_bn(c1p_g, c1p_b, c1p_m, c1p_v)
    s_c1u, t_c1u = _fold_bn(c1u_g, c1u_b, c1u_m, c1u_v)
    s_c2p, t_c2p = _fold_bn(c2p_g, c2p_b, c2p_m, c2p_v)
    s_c2u, t_c2u = _fold_bn(c2u_g, c2u_b, c2u_m, c2u_v)
    s_post, t_post = _fold_bn(post_g, post_b, post_m, post_v)

    zrows = jnp.zeros((NPAD,), _f32)
    zcnt = jnp.zeros((NPAD,), _f32)

    x, y1 = _tc_pre(node_features, s_pre, t_pre, pre_w, pre_bias,
                    s_c1p, t_c1p, c1p_w, c1p_bias)

    ps1, pc = _agg_cnt(y1, dstp, srcp, zrows, zcnt)
    ps1 = ps1.reshape(D, NPAD)
    pcr = pc[:N, None]

    x1, y2 = _tc_upd(x, ps1, pcr,
                     s_c1u[:H], t_c1u[:H], s_c1u[H:], t_c1u[H:],
                     c1u_w[:H], c1u_w[H:], c1u_bias,
                     s_c2p, t_c2p, c2p_w, c2p_bias)

    (ps2,) = _agg(y2, dstp, srcp, zrows)
    ps2 = ps2.reshape(D, NPAD)

    ow_pad = jnp.zeros((H, H), _f32).at[:, :C].set(out_w)
    ob_pad = jnp.zeros((H,), _f32).at[:C].set(out_b)
    (z,) = _tc_fin(x1, ps2, pcr,
                   s_c2u[:H], t_c2u[:H], s_c2u[H:], t_c2u[H:],
                   c2u_w[:H], c2u_w[H:], c2u_bias,
                   s_post, t_post, post_w, post_bias,
                   ow_pad, ob_pad)

    idx = jnp.concatenate(
        [input_node_indices.astype(jnp.int32),
         jnp.zeros((NPAD - N,), jnp.int32)])
    return _final_gather(z, idx)[:N, :C]


# speedup vs baseline: 3.6815x; 3.6252x over previous
import functools

import jax
import jax.numpy as jnp
from jax import lax
from jax.experimental import pallas as pl
from jax.experimental.pallas import tpu as pltpu
from jax.experimental.pallas import tpu_sc as plsc

N = 10000
E = 320000
D = 128
H = 128
C = 16

NC = 2
NS = 16
NW = NC * NS
EPT = E // NW
K = 16
GK = 5
NG = EPT // (K * GK)
NPAD = 10240
RPT = NPAD // NS
CPT = NPAD // NS
BPW = NPAD // NW

_SQRT_HALF = 0.7071067811865476


def _gelu(x):
    return x * 0.5 * (1.0 + lax.erf(x * _SQRT_HALF))


BM = 1024


def _dot(a, b):
    return jnp.dot(a, b, preferred_element_type=jnp.float32,
                   precision=lax.Precision.HIGHEST)


def _tc_pre_body(nf, s1, t1, w1, b1, s2, t2, w2, b2, x_out, y_out):
    x = _gelu(_dot(nf[...] * s1[...] + t1[...], w1[...]) + b1[...])
    x_out[...] = x
    y = _gelu(_dot(x * s2[...] + t2[...], w2[...]) + b2[...])
    y_out[...] = y.T.reshape(32, 4, y.shape[0])


def _tc_upd_body(x, ps, pc, sa, ta, sb, tb, wa, wb, bu, s2, t2, w2, b2,
                 x1_out, y2_out):
    sums = ps[...].T
    cnt = jnp.maximum(pc[...], 1.0)
    agg = sums / cnt
    h = (_dot(x[...] * sa[...] + ta[...], wa[...])
         + _dot(agg * sb[...] + tb[...], wb[...]) + bu[...])
    x1 = _gelu(h)
    x1_out[...] = x1
    y2 = _gelu(_dot(x1 * s2[...] + t2[...], w2[...]) + b2[...])
    y2_out[...] = y2.T.reshape(32, 4, y2.shape[0])


def _tc_fin_body(x, ps, pc, sa, ta, sb, tb, wa, wb, bu, sp, tp, wp, bp,
                 ow, ob, z_out):
    sums = ps[...].T
    cnt = jnp.maximum(pc[...], 1.0)
    agg = sums / cnt
    h = (_dot(x[...] * sa[...] + ta[...], wa[...])
         + _dot(agg * sb[...] + tb[...], wb[...]) + bu[...])
    x2 = _gelu(h)
    xp = _gelu(_dot(x2 * sp[...] + tp[...], wp[...]) + bp[...])
    z_out[...] = _dot(xp, ow[...]) + ob[...]


def _vec_spec(n):
    return pl.BlockSpec((n,), lambda i: (0,))


def _mat_spec(r, c):
    return pl.BlockSpec((r, c), lambda i: (0, 0))


def _row_spec(c):
    return pl.BlockSpec((BM, c), lambda i: (i, 0))


def _ps_spec():
    return pl.BlockSpec((D, BM), lambda i: (0, i))


def _pc_spec():
    return pl.BlockSpec((BM, 1), lambda i: (i, 0))


def _yt_spec():
    return pl.BlockSpec((32, 4, BM), lambda i: (0, 0, i))


_f32 = jnp.float32

_tc_pre = pl.pallas_call(
    _tc_pre_body,
    grid=(NPAD // BM,),
    in_specs=[_row_spec(D),
              _vec_spec(D), _vec_spec(D), _mat_spec(D, H), _vec_spec(H),
              _vec_spec(H), _vec_spec(H), _mat_spec(H, H), _vec_spec(H)],
    out_specs=[_row_spec(H), _yt_spec()],
    out_shape=[jax.ShapeDtypeStruct((N, H), _f32),
               jax.ShapeDtypeStruct((32, 4, NPAD), _f32)],
)

_tc_upd = pl.pallas_call(
    _tc_upd_body,
    grid=(NPAD // BM,),
    in_specs=[_row_spec(H), _ps_spec(), _pc_spec(),
              _vec_spec(H), _vec_spec(H), _vec_spec(H), _vec_spec(H),
              _mat_spec(H, H), _mat_spec(H, H), _vec_spec(H),
              _vec_spec(H), _vec_spec(H), _mat_spec(H, H), _vec_spec(H)],
    out_specs=[_row_spec(H), _yt_spec()],
    out_shape=[jax.ShapeDtypeStruct((N, H), _f32),
               jax.ShapeDtypeStruct((32, 4, NPAD), _f32)],
)

_tc_fin = pl.pallas_call(
    _tc_fin_body,
    grid=(NPAD // BM,),
    in_specs=[_row_spec(H), _ps_spec(), _pc_spec(),
              _vec_spec(H), _vec_spec(H), _vec_spec(H), _vec_spec(H),
              _mat_spec(H, H), _mat_spec(H, H), _vec_spec(H),
              _vec_spec(H), _vec_spec(H), _mat_spec(H, H), _vec_spec(H),
              _mat_spec(H, H), _vec_spec(H)],
    out_specs=[_row_spec(H)],
    out_shape=[jax.ShapeDtypeStruct((N, H), _f32)],
)


_MESH = plsc.VectorSubcoreMesh(core_axis_name="c", subcore_axis_name="s")

EP = 327680
EPT2 = EP // NW
KG = 80
NCH = EPT2 // KG
NPAIR = NCH // 2
SUB = KG // 16
DRT = EPT2 // 16
KF = 80


ECH = 2048
NECH = EP // ECH
UNR = 4


def _make_agg(with_cnt):
    out_type = [jax.ShapeDtypeStruct((NW, 4, NPAD), _f32)]
    scratch = [
        pltpu.VMEM((4 * NPAD,), _f32),
        pltpu.VMEM((4 * NPAD,), _f32),
        pltpu.VMEM((ECH,), jnp.int32),
        pltpu.VMEM((ECH,), jnp.int32),
        pltpu.VMEM((ECH,), jnp.int32),
        pltpu.VMEM((ECH,), jnp.int32),
        pltpu.SemaphoreType.DMA,
        pltpu.SemaphoreType.DMA,
    ]
    if with_cnt:
        out_type.append(jax.ShapeDtypeStruct((NPAD,), _f32))
        scratch.append(pltpu.VMEM((NPAD,), _f32))

    def body(*refs):
        if with_cnt:
            (y_hbm, dst_hbm, src_hbm, zrows_hbm, zcnt_hbm,
             sum_out, cnt_out, y_loc, acc, sa_, sb_, da_, db_,
             sem_a, sem_b, cacc) = refs
        else:
            (y_hbm, dst_hbm, src_hbm, zrows_hbm,
             sum_out, y_loc, acc, sa_, sb_, da_, db_,
             sem_a, sem_b) = refs
        cid = lax.axis_index("c")
        sid = lax.axis_index("s")
        wid = cid * NS + sid

        for c in range(4):
            pltpu.sync_copy(y_hbm.at[wid, c], y_loc.at[pl.ds(c * NPAD, NPAD)])
            pltpu.sync_copy(zrows_hbm, acc.at[pl.ds(c * NPAD, NPAD)])
        if with_cnt:
            pltpu.sync_copy(zcnt_hbm, cacc)
        ones16 = jnp.ones((16,), _f32)
        coff = [jnp.full((16,), c * NPAD, jnp.int32) for c in range(4)]

        def fire(c, sbuf, dbuf, sem):
            pltpu.async_copy(src_hbm.at[pl.ds(c * ECH, ECH)], sbuf, sem)
            pltpu.async_copy(dst_hbm.at[pl.ds(c * ECH, ECH)], dbuf, sem)

        def drain(sbuf, dbuf, sem):
            pltpu.make_async_copy(src_hbm.at[pl.ds(0, ECH)], sbuf, sem).wait()
            pltpu.make_async_copy(dst_hbm.at[pl.ds(0, ECH)], dbuf, sem).wait()

        def process(sbuf, dbuf):
            @functools.partial(plsc.parallel_loop, 0, ECH // 16, unroll=UNR)
            def _step(i):
                s16 = sbuf[pl.ds(i * 16, 16)]
                d16 = dbuf[pl.ds(i * 16, 16)]
                for c in range(4):
                    v = plsc.load_gather(y_loc, [s16 + coff[c]])
                    plsc.addupdate_scatter(acc, [d16 + coff[c]], v)
                if with_cnt:
                    plsc.addupdate_scatter(cacc, [d16], ones16)

        fire(0, sa_, da_, sem_a)

        def pair(g, carry):
            cA = 2 * g
            fire(cA + 1, sb_, db_, sem_b)
            drain(sa_, da_, sem_a)
            process(sa_, da_)

            @pl.when(g < NECH // 2 - 1)
            def _():
                fire(cA + 2, sa_, da_, sem_a)

            drain(sb_, db_, sem_b)
            process(sb_, db_)
            return carry

        lax.fori_loop(0, NECH // 2, pair, 0)

        for c in range(4):
            pltpu.sync_copy(acc.at[pl.ds(c * NPAD, NPAD)], sum_out.at[wid, c])
        if with_cnt:
            pltpu.sync_copy(cacc.at[pl.ds(wid * BPW, BPW)],
                            cnt_out.at[pl.ds(wid * BPW, BPW)])

    return pl.kernel(body, out_type=out_type, mesh=_MESH,
                     scratch_types=scratch,
                     compiler_params=pltpu.CompilerParams(
                         use_tc_tiling_on_sc=False,
                         needs_layout_passes=False))


_agg_cnt = _make_agg(True)
_agg = _make_agg(False)


@functools.partial(
    pl.kernel,
    out_type=jax.ShapeDtypeStruct((NPAD, H), _f32),
    mesh=_MESH,
    scratch_types=[
        pltpu.VMEM((BPW,), jnp.int32),
        pltpu.VMEM((BPW // KF, KF, H), _f32),
        pltpu.SemaphoreType.DMA,
    ],
)
def _final_gather(z_hbm, idx_hbm, out_hbm, idx_v, rows_v, sem):
    wid = lax.axis_index("c") * NS + lax.axis_index("s")
    base = wid * BPW
    pltpu.sync_copy(idx_hbm.at[pl.ds(base, BPW)], idx_v)
    descs = []
    for j in range(BPW // KF):
        descs.append(
            pltpu.async_copy(z_hbm.at[idx_v.at[pl.ds(j * KF, KF)]],
                             rows_v.at[j], sem))
    for j in range(BPW // KF):
        descs[j].wait()
        pltpu.sync_copy(rows_v.at[j], out_hbm.at[pl.ds(base + j * KF, KF)])


def _fold_bn(g, b, m, v):
    s = g / jnp.sqrt(v + 1e-5)
    return s, b - m * s


def kernel(input_node_indices, node_features, edge_index,
           pre_g, pre_b, pre_m, pre_v, pre_w, pre_bias,
           c1p_g, c1p_b, c1p_m, c1p_v, c1p_w, c1p_bias,
           c1u_g, c1u_b, c1u_m, c1u_v, c1u_w, c1u_bias,
           c2p_g, c2p_b, c2p_m, c2p_v, c2p_w, c2p_bias,
           c2u_g, c2u_b, c2u_m, c2u_v, c2u_w, c2u_bias,
           post_g, post_b, post_m, post_v, post_w, post_bias,
           out_w, out_b):
    dst = edge_index[0]
    src = edge_index[1]
    dstp = jnp.concatenate([dst, jnp.full((EP - E,), N + 200, jnp.int32)])
    srcp = jnp.concatenate([src, jnp.zeros((EP - E,), jnp.int32)])

    s_pre, t_pre = _fold_bn(pre_g, pre_b, pre_m, pre_v)
    s_c1p, t_c1p = _fold_bn(c1p_g, c1p_b, c1p_m, c1p_v)
    s_c1u, t_c1u = _fold_bn(c1u_g, c1u_b, c1u_m, c1u_v)
    s_c2p, t_c2p = _fold_bn(c2p_g, c2p_b, c2p_m, c2p_v)
    s_c2u, t_c2u = _fold_bn(c2u_g, c2u_b, c2u_m, c2u_v)
    s_post, t_post = _fold_bn(post_g, post_b, post_m, post_v)

    zrows = jnp.zeros((NPAD,), _f32)
    zcnt = jnp.zeros((NPAD,), _f32)

    x, y1 = _tc_pre(node_features, s_pre, t_pre, pre_w, pre_bias,
                    s_c1p, t_c1p, c1p_w, c1p_bias)

    ps1, pc = _agg_cnt(y1, dstp, srcp, zrows, zcnt)
    ps1 = ps1.reshape(D, NPAD)
    pcr = pc[:N, None]

    x1, y2 = _tc_upd(x, ps1, pcr,
                     s_c1u[:H], t_c1u[:H], s_c1u[H:], t_c1u[H:],
                     c1u_w[:H], c1u_w[H:], c1u_bias,
                     s_c2p, t_c2p, c2p_w, c2p_bias)

    (ps2,) = _agg(y2, dstp, srcp, zrows)
    ps2 = ps2.reshape(D, NPAD)

    ow_pad = jnp.zeros((H, H), _f32).at[:, :C].set(out_w)
    ob_pad = jnp.zeros((H,), _f32).at[:C].set(out_b)
    (z,) = _tc_fin(x1, ps2, pcr,
                   s_c2u[:H], t_c2u[:H], s_c2u[H:], t_c2u[H:],
                   c2u_w[:H], c2u_w[H:], c2u_bias,
                   s_post, t_post, post_w, post_bias,
                   ow_pad, ob_pad)

    idx = jnp.concatenate(
        [input_node_indices.astype(jnp.int32),
         jnp.zeros((NPAD - N,), jnp.int32)])
    return _final_gather(z, idx)[:N, :C]
